# Initial kernel scaffold; baseline (speedup 1.0000x reference)
import jax
import jax.numpy as jnp
from jax.experimental import pallas as pl


def kernel(basic_block, edge_index, h0, c0, W_gcn, b_gcn, W_ih, W_hh, b_ih, b_hh):
    raise NotImplementedError("write your pallas kernel here")


# speedup vs baseline: 8.5759x; 8.5759x over previous
import functools

import jax
import jax.numpy as jnp
from jax import lax
from jax.experimental import pallas as pl
from jax.experimental.pallas import tpu as pltpu

N = 10000
E = 160000
D = 256
H = 256
G4 = 4 * H


def _y_body(x_ref, wg_ref, deg_ref, y_ref, dinv_ref):
    deg = deg_ref[...] + 1.0
    dinv = lax.rsqrt(deg)
    xw = jnp.dot(x_ref[...], wg_ref[...], preferred_element_type=jnp.float32)
    y_ref[...] = xw * dinv
    dinv_ref[...] = dinv


def _compute_y(x, wg, deg_edges):
    BR = 1000
    grid = (N // BR,)
    return pl.pallas_call(
        _y_body,
        grid=grid,
        in_specs=[
            pl.BlockSpec((BR, D), lambda i: (i, 0)),
            pl.BlockSpec((D, H), lambda i: (0, 0)),
            pl.BlockSpec((BR, 1), lambda i: (i, 0)),
        ],
        out_specs=[
            pl.BlockSpec((BR, H), lambda i: (i, 0)),
            pl.BlockSpec((BR, 1), lambda i: (i, 0)),
        ],
        out_shape=[
            jax.ShapeDtypeStruct((N, H), jnp.float32),
            jax.ShapeDtypeStruct((N, 1), jnp.float32),
        ],
    )(x, wg, deg_edges)


def _gx_body(acc_ref, y_ref, dinv_ref, bg_ref, wt_ref, bias_ref, gx_ref):
    xg = dinv_ref[...] * (acc_ref[...] + y_ref[...]) + bg_ref[...]
    gx_ref[...] = (
        jnp.dot(xg, wt_ref[...], preferred_element_type=jnp.float32) + bias_ref[...]
    )


def _compute_gx(accum, y, dinv, b_gcn, w_ihT, bias):
    BR = 1000
    grid = (N // BR,)
    return pl.pallas_call(
        _gx_body,
        grid=grid,
        in_specs=[
            pl.BlockSpec((BR, H), lambda i: (i, 0)),
            pl.BlockSpec((BR, H), lambda i: (i, 0)),
            pl.BlockSpec((BR, 1), lambda i: (i, 0)),
            pl.BlockSpec((1, H), lambda i: (0, 0)),
            pl.BlockSpec((H, G4), lambda i: (0, 0)),
            pl.BlockSpec((1, G4), lambda i: (0, 0)),
        ],
        out_specs=pl.BlockSpec((BR, G4), lambda i: (i, 0)),
        out_shape=jax.ShapeDtypeStruct((N, G4), jnp.float32),
    )(accum, y, dinv, b_gcn, w_ihT, bias)


def _lstm_body(gx_ref, whhT_ref, h0_ref, c0_ref, out_ref, cn_ref, h_s, c_s, *, t_blk):
    @pl.when(pl.program_id(0) == 0)
    def _init():
        h_s[...] = h0_ref[...]
        c_s[...] = c0_ref[...]

    def step(t, carry):
        h, c = carry
        g = (
            jnp.dot(h, whhT_ref[...], preferred_element_type=jnp.float32)
            + gx_ref[pl.ds(t, 1), :]
        )
        i = jax.nn.sigmoid(g[:, 0:H])
        f = jax.nn.sigmoid(g[:, H : 2 * H])
        gg = jnp.tanh(g[:, 2 * H : 3 * H])
        o = jax.nn.sigmoid(g[:, 3 * H : 4 * H])
        c_new = f * c + i * gg
        h_new = o * jnp.tanh(c_new)
        out_ref[pl.ds(t, 1), :] = h_new
        return (h_new, c_new)

    h, c = lax.fori_loop(0, t_blk, step, (h_s[...], c_s[...]))
    h_s[...] = h
    c_s[...] = c
    cn_ref[...] = c


def _run_lstm(gx, w_hhT, h0, c0):
    T_BLK = 400
    grid = (N // T_BLK,)
    return pl.pallas_call(
        functools.partial(_lstm_body, t_blk=T_BLK),
        grid=grid,
        in_specs=[
            pl.BlockSpec((T_BLK, G4), lambda i: (i, 0)),
            pl.BlockSpec((H, G4), lambda i: (0, 0)),
            pl.BlockSpec((1, H), lambda i: (0, 0)),
            pl.BlockSpec((1, H), lambda i: (0, 0)),
        ],
        out_specs=[
            pl.BlockSpec((T_BLK, H), lambda i: (i, 0)),
            pl.BlockSpec((1, H), lambda i: (0, 0)),
        ],
        out_shape=[
            jax.ShapeDtypeStruct((N, H), jnp.float32),
            jax.ShapeDtypeStruct((1, H), jnp.float32),
        ],
        scratch_shapes=[
            pltpu.VMEM((1, H), jnp.float32),
            pltpu.VMEM((1, H), jnp.float32),
        ],
    )(gx, w_hhT, h0, c0)


def kernel(basic_block, edge_index, h0, c0, W_gcn, b_gcn, W_ih, W_hh, b_ih, b_hh):
    src = edge_index[0]
    dst = edge_index[1]

    deg_edges = jax.ops.segment_sum(
        jnp.ones((E,), jnp.float32), dst, num_segments=N
    ).reshape(N, 1)

    y, dinv = _compute_y(basic_block, W_gcn, deg_edges)

    accum = jax.ops.segment_sum(y[src], dst, num_segments=N)

    bias = (b_ih + b_hh).reshape(1, G4)
    gx = _compute_gx(accum, y, dinv, b_gcn.reshape(1, H), W_ih.T, bias)

    outs, c_n = _run_lstm(gx, W_hh.T, h0[0], c0[0])
    output = outs[:, None, :]
    h_n = outs[N - 1 :][None]
    c_n = c_n[None]
    return output, h_n, c_n


# speedup vs baseline: 11.4715x; 1.3377x over previous
import functools

import jax
import jax.numpy as jnp
from jax import lax
from jax.experimental import pallas as pl
from jax.experimental.pallas import tpu as pltpu
from jax.experimental.pallas import tpu_sc as plsc

N = 10000
E = 160000
D = 256
H = 256
G4 = 4 * H

_SC_MESH = plsc.VectorSubcoreMesh(core_axis_name="c", subcore_axis_name="s")
_NSC = 2
_NTILE = 16
_HALF = N // _NSC


_DR = 48
_EPW = E // (_NSC * _NTILE)
_DCH = 1000


def _deg_body(dst_hbm, zeros_hbm, out_hbm, idx_v, cnt_v, cnt2d_v, iota_v, shared):
    cid = lax.axis_index("c")
    sid = lax.axis_index("s")
    w = sid * _NSC + cid
    zeros16 = jnp.zeros((16,), jnp.float32)

    def zvec(k, carry):
        cnt_v[pl.ds(16 * k, 16)] = zeros16
        return carry

    lax.fori_loop(0, _DR * 16, zvec, 0)

    @pl.when(sid == 0)
    def _zero_shared():
        pltpu.sync_copy(zeros_hbm, shared)

    for j in range(_DR // 16):
        iota_v[pl.ds(16 * j, 16)] = lax.iota(jnp.int32, 16) + 16 * j
    plsc.subcore_barrier()

    ones = jnp.full((16,), 1.0, jnp.float32)
    tail_mask = lax.iota(jnp.int32, 16) < (_DCH % 16 or 16)
    base = w * _EPW

    def chunk(c, carry):
        pltpu.sync_copy(dst_hbm.at[pl.ds(base + c * _DCH, _DCH)],
                        idx_v.at[pl.ds(0, _DCH)])

        def vec(j, carry2):
            d = idx_v[pl.ds(16 * j, 16)]
            plsc.addupdate_scatter(cnt_v, [d], ones)
            return carry2

        lax.fori_loop(0, _DCH // 16, vec, 0)
        d = idx_v[pl.ds((_DCH // 16) * 16, 16)]
        plsc.addupdate_scatter(cnt_v, [d], ones, mask=tail_mask)
        return carry

    lax.fori_loop(0, _EPW // _DCH, chunk, 0)

    def pack(k, carry):
        r = k >> 4
        j = k & 15
        cnt2d_v[r, pl.ds(16 * j, 16)] = cnt_v[pl.ds(16 * k, 16)]
        return carry

    lax.fori_loop(0, _DR * 16, pack, 0)
    pltpu.sync_copy(cnt2d_v, shared.at[iota_v], add=True)
    plsc.subcore_barrier()

    @pl.when(sid == 0)
    def _out():
        pltpu.sync_copy(shared, out_hbm.at[cid])


def _compute_deg(dst):
    zeros = jnp.zeros((_DR, 256), jnp.float32)
    f = pl.kernel(
        _deg_body,
        out_type=jax.ShapeDtypeStruct((_NSC, _DR, 256), jnp.float32),
        mesh=_SC_MESH,
        compiler_params=pltpu.CompilerParams(use_tc_tiling_on_sc=False, needs_layout_passes=False),
        scratch_types=[
            pltpu.VMEM((_DCH + 8, ), jnp.int32),
            pltpu.VMEM((_DR * 256,), jnp.float32),
            pltpu.VMEM((_DR, 256), jnp.float32),
            pltpu.VMEM((_DR,), jnp.int32),
            pltpu.VMEM_SHARED((_DR, 256), jnp.float32),
        ],
    )
    return f(dst, zeros)


_ACC_ROWS = _HALF + 8
_MCH = 80
_EPT = E // _NTILE
_ORC = 320


def _msg_body(y_hbm, src_hbm, dst_hbm, zeros_hbm, out_hbm,
              srcv, dstv, dlocv, rows_v, shared):
    cid = lax.axis_index("c")
    sid = lax.axis_index("s")
    lo = cid * _HALF
    nz = _HALF - 15 * _ORC

    @pl.when(sid < 15)
    def _z0():
        pltpu.sync_copy(zeros_hbm, shared.at[pl.ds(sid * _ORC, _ORC)])

    @pl.when(sid == 15)
    def _z1():
        pltpu.sync_copy(zeros_hbm.at[pl.ds(0, nz)],
                        shared.at[pl.ds(15 * _ORC, nz)])

    plsc.subcore_barrier()
    base = sid * _EPT

    def chunk(c, carry):
        off = base + c * _MCH
        pltpu.sync_copy(src_hbm.at[pl.ds(off, _MCH)], srcv)
        pltpu.sync_copy(dst_hbm.at[pl.ds(off, _MCH)], dstv)

        def vec(j, carry2):
            d = dstv[pl.ds(16 * j, 16)]
            inb = (d >= lo) & (d < lo + _HALF)
            dlocv[pl.ds(16 * j, 16)] = jnp.where(inb, d - lo, _HALF)
            return carry2

        lax.fori_loop(0, _MCH // 16, vec, 0)
        pltpu.sync_copy(y_hbm.at[srcv], rows_v)
        pltpu.sync_copy(rows_v, shared.at[dlocv], add=True)
        return carry

    lax.fori_loop(0, _EPT // _MCH, chunk, 0)
    plsc.subcore_barrier()

    @pl.when(sid < 15)
    def _o0():
        pltpu.sync_copy(shared.at[pl.ds(sid * _ORC, _ORC)],
                        out_hbm.at[pl.ds(lo + sid * _ORC, _ORC)])

    @pl.when(sid == 15)
    def _o1():
        pltpu.sync_copy(shared.at[pl.ds(15 * _ORC, nz)],
                        out_hbm.at[pl.ds(lo + 15 * _ORC, nz)])


def _compute_accum(y, src, dst):
    zeros = jnp.zeros((_ORC, 256), jnp.float32)
    f = pl.kernel(
        _msg_body,
        out_type=jax.ShapeDtypeStruct((N, H), jnp.float32),
        mesh=_SC_MESH,
        compiler_params=pltpu.CompilerParams(use_tc_tiling_on_sc=False, needs_layout_passes=False),
        scratch_types=[
            pltpu.VMEM((_MCH,), jnp.int32),
            pltpu.VMEM((_MCH,), jnp.int32),
            pltpu.VMEM((_MCH,), jnp.int32),
            pltpu.VMEM((_MCH, H), jnp.float32),
            pltpu.VMEM_SHARED((_ACC_ROWS, H), jnp.float32),
        ],
    )
    return f(y, src, dst, zeros)


def _y_body(x_ref, wg_ref, deg0_ref, deg1_ref, y_ref, dinv_ref):
    deg = deg0_ref[...] + deg1_ref[...] + 1.0
    dinv = lax.rsqrt(deg)
    xw = jnp.dot(x_ref[...], wg_ref[...], preferred_element_type=jnp.float32)
    y_ref[...] = xw * dinv
    dinv_ref[...] = dinv


def _compute_y(x, wg, deg0, deg1):
    BR = 1000
    grid = (N // BR,)
    return pl.pallas_call(
        _y_body,
        grid=grid,
        in_specs=[
            pl.BlockSpec((BR, D), lambda i: (i, 0)),
            pl.BlockSpec((D, H), lambda i: (0, 0)),
            pl.BlockSpec((BR, 1), lambda i: (i, 0)),
            pl.BlockSpec((BR, 1), lambda i: (i, 0)),
        ],
        out_specs=[
            pl.BlockSpec((BR, H), lambda i: (i, 0)),
            pl.BlockSpec((BR, 1), lambda i: (i, 0)),
        ],
        out_shape=[
            jax.ShapeDtypeStruct((N, H), jnp.float32),
            jax.ShapeDtypeStruct((N, 1), jnp.float32),
        ],
    )(x, wg, deg0, deg1)


def _gx_body(acc_ref, y_ref, dinv_ref, bg_ref, wt_ref, bias_ref, gx_ref):
    xg = dinv_ref[...] * (acc_ref[...] + y_ref[...]) + bg_ref[...]
    gx_ref[...] = (
        jnp.dot(xg, wt_ref[...], preferred_element_type=jnp.float32) + bias_ref[...]
    )


def _compute_gx(accum, y, dinv, b_gcn, w_ihT, bias):
    BR = 1000
    grid = (N // BR,)
    return pl.pallas_call(
        _gx_body,
        grid=grid,
        in_specs=[
            pl.BlockSpec((BR, H), lambda i: (i, 0)),
            pl.BlockSpec((BR, H), lambda i: (i, 0)),
            pl.BlockSpec((BR, 1), lambda i: (i, 0)),
            pl.BlockSpec((1, H), lambda i: (0, 0)),
            pl.BlockSpec((H, G4), lambda i: (0, 0)),
            pl.BlockSpec((1, G4), lambda i: (0, 0)),
        ],
        out_specs=pl.BlockSpec((BR, G4), lambda i: (i, 0)),
        out_shape=jax.ShapeDtypeStruct((N, G4), jnp.float32),
    )(accum, y, dinv, b_gcn, w_ihT, bias)


def _lstm_body(gx_ref, whhT_ref, h0_ref, c0_ref, out_ref, cn_ref, h_s, c_s, *, t_blk):
    @pl.when(pl.program_id(0) == 0)
    def _init():
        h_s[...] = h0_ref[...]
        c_s[...] = c0_ref[...]

    def step(t, carry):
        h, c = carry
        g = (
            jnp.dot(h, whhT_ref[...], preferred_element_type=jnp.float32)
            + gx_ref[pl.ds(t, 1), :]
        )
        i = jax.nn.sigmoid(g[:, 0:H])
        f = jax.nn.sigmoid(g[:, H : 2 * H])
        gg = jnp.tanh(g[:, 2 * H : 3 * H])
        o = jax.nn.sigmoid(g[:, 3 * H : 4 * H])
        c_new = f * c + i * gg
        h_new = o * jnp.tanh(c_new)
        out_ref[pl.ds(t, 1), :] = h_new
        return (h_new, c_new)

    h, c = lax.fori_loop(0, t_blk, step, (h_s[...], c_s[...]))
    h_s[...] = h
    c_s[...] = c
    cn_ref[...] = c


def _run_lstm(gx, w_hhT, h0, c0):
    T_BLK = 400
    grid = (N // T_BLK,)
    return pl.pallas_call(
        functools.partial(_lstm_body, t_blk=T_BLK),
        grid=grid,
        in_specs=[
            pl.BlockSpec((T_BLK, G4), lambda i: (i, 0)),
            pl.BlockSpec((H, G4), lambda i: (0, 0)),
            pl.BlockSpec((1, H), lambda i: (0, 0)),
            pl.BlockSpec((1, H), lambda i: (0, 0)),
        ],
        out_specs=[
            pl.BlockSpec((T_BLK, H), lambda i: (i, 0)),
            pl.BlockSpec((1, H), lambda i: (0, 0)),
        ],
        out_shape=[
            jax.ShapeDtypeStruct((N, H), jnp.float32),
            jax.ShapeDtypeStruct((1, H), jnp.float32),
        ],
        scratch_shapes=[
            pltpu.VMEM((1, H), jnp.float32),
            pltpu.VMEM((1, H), jnp.float32),
        ],
    )(gx, w_hhT, h0, c0)


def kernel(basic_block, edge_index, h0, c0, W_gcn, b_gcn, W_ih, W_hh, b_ih, b_hh):
    src = edge_index[0]
    dst = edge_index[1]

    degp = _compute_deg(dst)
    degp = degp.reshape(_NSC, _DR * 256)
    deg0 = degp[0, :N].reshape(N, 1)
    deg1 = degp[1, :N].reshape(N, 1)

    y, dinv = _compute_y(basic_block, W_gcn, deg0, deg1)

    accum = _compute_accum(y, src, dst)

    bias = (b_ih + b_hh).reshape(1, G4)
    gx = _compute_gx(accum, y, dinv, b_gcn.reshape(1, H), W_ih.T, bias)

    outs, c_n = _run_lstm(gx, W_hh.T, h0[0], c0[0])
    output = outs[:, None, :]
    h_n = outs[N - 1 :][None]
    c_n = c_n[None]
    return output, h_n, c_n


# speedup vs baseline: 11.4872x; 1.0014x over previous
import functools

import jax
import jax.numpy as jnp
from jax import lax
from jax.experimental import pallas as pl
from jax.experimental.pallas import tpu as pltpu
from jax.experimental.pallas import tpu_sc as plsc

N = 10000
E = 160000
D = 256
H = 256
G4 = 4 * H

_SC_MESH = plsc.VectorSubcoreMesh(core_axis_name="c", subcore_axis_name="s")
_NSC = 2
_NTILE = 16
_HALF = N // _NSC


_DR = 48
_EPW = E // (_NSC * _NTILE)
_DCH = 1000


def _deg_body(dst_hbm, zeros_hbm, out_hbm, idx_v, cnt_v, cnt2d_v, iota_v, shared):
    cid = lax.axis_index("c")
    sid = lax.axis_index("s")
    w = sid * _NSC + cid
    zeros16 = jnp.zeros((16,), jnp.float32)

    def zvec(k, carry):
        cnt_v[pl.ds(16 * k, 16)] = zeros16
        return carry

    lax.fori_loop(0, _DR * 16, zvec, 0)

    @pl.when(sid == 0)
    def _zero_shared():
        pltpu.sync_copy(zeros_hbm, shared)

    for j in range(_DR // 16):
        iota_v[pl.ds(16 * j, 16)] = lax.iota(jnp.int32, 16) + 16 * j
    plsc.subcore_barrier()

    ones = jnp.full((16,), 1.0, jnp.float32)
    tail_mask = lax.iota(jnp.int32, 16) < (_DCH % 16 or 16)
    base = w * _EPW

    def chunk(c, carry):
        pltpu.sync_copy(dst_hbm.at[pl.ds(base + c * _DCH, _DCH)],
                        idx_v.at[pl.ds(0, _DCH)])

        def vec(j, carry2):
            d = idx_v[pl.ds(16 * j, 16)]
            plsc.addupdate_scatter(cnt_v, [d], ones)
            return carry2

        lax.fori_loop(0, _DCH // 16, vec, 0)
        d = idx_v[pl.ds((_DCH // 16) * 16, 16)]
        plsc.addupdate_scatter(cnt_v, [d], ones, mask=tail_mask)
        return carry

    lax.fori_loop(0, _EPW // _DCH, chunk, 0)

    def pack(k, carry):
        r = k >> 4
        j = k & 15
        cnt2d_v[r, pl.ds(16 * j, 16)] = cnt_v[pl.ds(16 * k, 16)]
        return carry

    lax.fori_loop(0, _DR * 16, pack, 0)
    pltpu.sync_copy(cnt2d_v, shared.at[iota_v], add=True)
    plsc.subcore_barrier()

    @pl.when(sid == 0)
    def _out():
        pltpu.sync_copy(shared, out_hbm.at[cid])


def _compute_deg(dst):
    zeros = jnp.zeros((_DR, 256), jnp.float32)
    f = pl.kernel(
        _deg_body,
        out_type=jax.ShapeDtypeStruct((_NSC, _DR, 256), jnp.float32),
        mesh=_SC_MESH,
        compiler_params=pltpu.CompilerParams(use_tc_tiling_on_sc=False, needs_layout_passes=False),
        scratch_types=[
            pltpu.VMEM((_DCH + 8, ), jnp.int32),
            pltpu.VMEM((_DR * 256,), jnp.float32),
            pltpu.VMEM((_DR, 256), jnp.float32),
            pltpu.VMEM((_DR,), jnp.int32),
            pltpu.VMEM_SHARED((_DR, 256), jnp.float32),
        ],
    )
    return f(dst, zeros)


_ACC_ROWS = _HALF + 8
_MCH = 80
_EPT = E // _NTILE
_ORC = 320


def _msg_body(y_hbm, src_hbm, dst_hbm, zeros_hbm, out_hbm,
              srcv, dstv, dlocv, rows_v, shared):
    cid = lax.axis_index("c")
    sid = lax.axis_index("s")
    lo = cid * _HALF
    nz = _HALF - 15 * _ORC

    @pl.when(sid < 15)
    def _z0():
        pltpu.sync_copy(zeros_hbm, shared.at[pl.ds(sid * _ORC, _ORC)])

    @pl.when(sid == 15)
    def _z1():
        pltpu.sync_copy(zeros_hbm.at[pl.ds(0, nz)],
                        shared.at[pl.ds(15 * _ORC, nz)])

    plsc.subcore_barrier()
    base = sid * _EPT

    def chunk(c, carry):
        off = base + c * _MCH
        pltpu.sync_copy(src_hbm.at[pl.ds(off, _MCH)], srcv)
        pltpu.sync_copy(dst_hbm.at[pl.ds(off, _MCH)], dstv)

        def vec(j, carry2):
            d = dstv[pl.ds(16 * j, 16)]
            inb = (d >= lo) & (d < lo + _HALF)
            dlocv[pl.ds(16 * j, 16)] = jnp.where(inb, d - lo, _HALF)
            return carry2

        lax.fori_loop(0, _MCH // 16, vec, 0)
        pltpu.sync_copy(y_hbm.at[srcv], rows_v)
        pltpu.sync_copy(rows_v, shared.at[dlocv], add=True)
        return carry

    lax.fori_loop(0, _EPT // _MCH, chunk, 0)
    plsc.subcore_barrier()

    @pl.when(sid < 15)
    def _o0():
        pltpu.sync_copy(shared.at[pl.ds(sid * _ORC, _ORC)],
                        out_hbm.at[pl.ds(lo + sid * _ORC, _ORC)])

    @pl.when(sid == 15)
    def _o1():
        pltpu.sync_copy(shared.at[pl.ds(15 * _ORC, nz)],
                        out_hbm.at[pl.ds(lo + 15 * _ORC, nz)])


def _compute_accum(y, src, dst):
    zeros = jnp.zeros((_ORC, 256), jnp.float32)
    f = pl.kernel(
        _msg_body,
        out_type=jax.ShapeDtypeStruct((N, H), jnp.float32),
        mesh=_SC_MESH,
        compiler_params=pltpu.CompilerParams(use_tc_tiling_on_sc=False, needs_layout_passes=False),
        scratch_types=[
            pltpu.VMEM((_MCH,), jnp.int32),
            pltpu.VMEM((_MCH,), jnp.int32),
            pltpu.VMEM((_MCH,), jnp.int32),
            pltpu.VMEM((_MCH, H), jnp.float32),
            pltpu.VMEM_SHARED((_ACC_ROWS, H), jnp.float32),
        ],
    )
    return f(y, src, dst, zeros)


def _y_body(x_ref, wg_ref, deg0_ref, deg1_ref, y_ref, dinv_ref):
    deg = deg0_ref[...] + deg1_ref[...] + 1.0
    dinv = lax.rsqrt(deg)
    xw = jnp.dot(x_ref[...], wg_ref[...], preferred_element_type=jnp.float32)
    y_ref[...] = xw * dinv
    dinv_ref[...] = dinv


def _compute_y(x, wg, deg0, deg1):
    BR = 1000
    grid = (N // BR,)
    return pl.pallas_call(
        _y_body,
        grid=grid,
        in_specs=[
            pl.BlockSpec((BR, D), lambda i: (i, 0)),
            pl.BlockSpec((D, H), lambda i: (0, 0)),
            pl.BlockSpec((BR, 1), lambda i: (i, 0)),
            pl.BlockSpec((BR, 1), lambda i: (i, 0)),
        ],
        out_specs=[
            pl.BlockSpec((BR, H), lambda i: (i, 0)),
            pl.BlockSpec((BR, 1), lambda i: (i, 0)),
        ],
        out_shape=[
            jax.ShapeDtypeStruct((N, H), jnp.float32),
            jax.ShapeDtypeStruct((N, 1), jnp.float32),
        ],
    )(x, wg, deg0, deg1)


def _gx_body(acc_ref, y_ref, dinv_ref, bg_ref, wt_ref, bias_ref, gx_ref):
    xg = dinv_ref[...] * (acc_ref[...] + y_ref[...]) + bg_ref[...]
    gx_ref[...] = (
        jnp.dot(xg, wt_ref[...], preferred_element_type=jnp.float32) + bias_ref[...]
    )


def _compute_gx(accum, y, dinv, b_gcn, w_ihT, bias):
    BR = 1000
    grid = (N // BR,)
    return pl.pallas_call(
        _gx_body,
        grid=grid,
        in_specs=[
            pl.BlockSpec((BR, H), lambda i: (i, 0)),
            pl.BlockSpec((BR, H), lambda i: (i, 0)),
            pl.BlockSpec((BR, 1), lambda i: (i, 0)),
            pl.BlockSpec((1, H), lambda i: (0, 0)),
            pl.BlockSpec((H, G4), lambda i: (0, 0)),
            pl.BlockSpec((1, G4), lambda i: (0, 0)),
        ],
        out_specs=pl.BlockSpec((BR, G4), lambda i: (i, 0)),
        out_shape=jax.ShapeDtypeStruct((N, G4), jnp.float32),
    )(accum, y, dinv, b_gcn, w_ihT, bias)


def _lstm_body(gx_ref, whhT_ref, wlo_ref, h0_ref, c0_ref, out_ref, cn_ref, h_s, c_s, *, t_blk):
    @pl.when(pl.program_id(0) == 0)
    def _init():
        h_s[...] = h0_ref[...]
        c_s[...] = c0_ref[...]

    def step(t, carry):
        h, c = carry
        hb = h.astype(jnp.bfloat16)
        g = (
            jnp.dot(hb, whhT_ref[...], preferred_element_type=jnp.float32)
            + gx_ref[pl.ds(t, 1), :]
        )
        i = jax.nn.sigmoid(g[:, 0:H])
        f = jax.nn.sigmoid(g[:, H : 2 * H])
        gg = jnp.tanh(g[:, 2 * H : 3 * H])
        o = jax.nn.sigmoid(g[:, 3 * H : 4 * H])
        c_new = f * c + i * gg
        h_new = o * jnp.tanh(c_new)
        out_ref[pl.ds(t, 1), :] = h_new
        return (h_new, c_new)

    h, c = lax.fori_loop(0, t_blk, step, (h_s[...], c_s[...]))
    h_s[...] = h
    c_s[...] = c
    cn_ref[...] = c


def _run_lstm(gx, w_hhT, h0, c0):
    w_hi = w_hhT.astype(jnp.bfloat16)
    w_lo = (w_hhT - w_hi.astype(jnp.float32)).astype(jnp.bfloat16)
    T_BLK = 400
    grid = (N // T_BLK,)
    return pl.pallas_call(
        functools.partial(_lstm_body, t_blk=T_BLK),
        grid=grid,
        in_specs=[
            pl.BlockSpec((T_BLK, G4), lambda i: (i, 0)),
            pl.BlockSpec((H, G4), lambda i: (0, 0)),
            pl.BlockSpec((H, G4), lambda i: (0, 0)),
            pl.BlockSpec((1, H), lambda i: (0, 0)),
            pl.BlockSpec((1, H), lambda i: (0, 0)),
        ],
        out_specs=[
            pl.BlockSpec((T_BLK, H), lambda i: (i, 0)),
            pl.BlockSpec((1, H), lambda i: (0, 0)),
        ],
        out_shape=[
            jax.ShapeDtypeStruct((N, H), jnp.float32),
            jax.ShapeDtypeStruct((1, H), jnp.float32),
        ],
        scratch_shapes=[
            pltpu.VMEM((1, H), jnp.float32),
            pltpu.VMEM((1, H), jnp.float32),
        ],
    )(gx, w_hi, w_lo, h0, c0)


def kernel(basic_block, edge_index, h0, c0, W_gcn, b_gcn, W_ih, W_hh, b_ih, b_hh):
    src = edge_index[0]
    dst = edge_index[1]

    degp = _compute_deg(dst)
    degp = degp.reshape(_NSC, _DR * 256)
    deg0 = degp[0, :N].reshape(N, 1)
    deg1 = degp[1, :N].reshape(N, 1)

    y, dinv = _compute_y(basic_block, W_gcn, deg0, deg1)

    accum = _compute_accum(y, src, dst)

    bias = (b_ih + b_hh).reshape(1, G4)
    gx = _compute_gx(accum, y, dinv, b_gcn.reshape(1, H), W_ih.T, bias)

    outs, c_n = _run_lstm(gx, W_hh.T, h0[0], c0[0])
    output = outs[:, None, :]
    h_n = outs[N - 1 :][None]
    c_n = c_n[None]
    return output, h_n, c_n


# speedup vs baseline: 13.3258x; 1.1601x over previous
import functools

import jax
import jax.numpy as jnp
from jax import lax
from jax.experimental import pallas as pl
from jax.experimental.pallas import tpu as pltpu
from jax.experimental.pallas import tpu_sc as plsc

N = 10000
E = 160000
D = 256
H = 256
G4 = 4 * H

_SC_MESH = plsc.VectorSubcoreMesh(core_axis_name="c", subcore_axis_name="s")
_NSC = 2
_NTILE = 16
_HALF = N // _NSC


_DR = 48
_EPW = E // (_NSC * _NTILE)
_DCH = 1000


def _deg_body(dst_hbm, zeros_hbm, out_hbm, idx_v, cnt_v, cnt2d_v, iota_v, shared):
    cid = lax.axis_index("c")
    sid = lax.axis_index("s")
    w = sid * _NSC + cid
    zeros16 = jnp.zeros((16,), jnp.float32)

    def zvec(k, carry):
        cnt_v[pl.ds(16 * k, 16)] = zeros16
        return carry

    lax.fori_loop(0, _DR * 16, zvec, 0)

    @pl.when(sid == 0)
    def _zero_shared():
        pltpu.sync_copy(zeros_hbm, shared)

    for j in range(_DR // 16):
        iota_v[pl.ds(16 * j, 16)] = lax.iota(jnp.int32, 16) + 16 * j
    plsc.subcore_barrier()

    ones = jnp.full((16,), 1.0, jnp.float32)
    tail_mask = lax.iota(jnp.int32, 16) < (_DCH % 16 or 16)
    base = w * _EPW

    def chunk(c, carry):
        pltpu.sync_copy(dst_hbm.at[pl.ds(base + c * _DCH, _DCH)],
                        idx_v.at[pl.ds(0, _DCH)])

        def vec(j, carry2):
            d = idx_v[pl.ds(16 * j, 16)]
            plsc.addupdate_scatter(cnt_v, [d], ones)
            return carry2

        lax.fori_loop(0, _DCH // 16, vec, 0)
        d = idx_v[pl.ds((_DCH // 16) * 16, 16)]
        plsc.addupdate_scatter(cnt_v, [d], ones, mask=tail_mask)
        return carry

    lax.fori_loop(0, _EPW // _DCH, chunk, 0)

    def pack(k, carry):
        r = k >> 4
        j = k & 15
        cnt2d_v[r, pl.ds(16 * j, 16)] = cnt_v[pl.ds(16 * k, 16)]
        return carry

    lax.fori_loop(0, _DR * 16, pack, 0)
    pltpu.sync_copy(cnt2d_v, shared.at[iota_v], add=True)
    plsc.subcore_barrier()

    @pl.when(sid == 0)
    def _out():
        pltpu.sync_copy(shared, out_hbm.at[cid])


def _compute_deg(dst):
    zeros = jnp.zeros((_DR, 256), jnp.float32)
    f = pl.kernel(
        _deg_body,
        out_type=jax.ShapeDtypeStruct((_NSC, _DR, 256), jnp.float32),
        mesh=_SC_MESH,
        compiler_params=pltpu.CompilerParams(use_tc_tiling_on_sc=False, needs_layout_passes=False),
        scratch_types=[
            pltpu.VMEM((_DCH + 8, ), jnp.int32),
            pltpu.VMEM((_DR * 256,), jnp.float32),
            pltpu.VMEM((_DR, 256), jnp.float32),
            pltpu.VMEM((_DR,), jnp.int32),
            pltpu.VMEM_SHARED((_DR, 256), jnp.float32),
        ],
    )
    return f(dst, zeros)


_ACC_ROWS = _HALF + 8
_MCH = 80
_EPT = E // _NTILE
_ORC = 320


def _msg_body(y_hbm, src_hbm, dst_hbm, zeros_hbm, out_hbm,
              srcv, dstv, dlocv, rows_v, shared):
    cid = lax.axis_index("c")
    sid = lax.axis_index("s")
    lo = cid * _HALF
    nz = _HALF - 15 * _ORC

    @pl.when(sid < 15)
    def _z0():
        pltpu.sync_copy(zeros_hbm, shared.at[pl.ds(sid * _ORC, _ORC)])

    @pl.when(sid == 15)
    def _z1():
        pltpu.sync_copy(zeros_hbm.at[pl.ds(0, nz)],
                        shared.at[pl.ds(15 * _ORC, nz)])

    plsc.subcore_barrier()
    base = sid * _EPT

    def chunk(c, carry):
        off = base + c * _MCH
        pltpu.sync_copy(src_hbm.at[pl.ds(off, _MCH)], srcv)
        pltpu.sync_copy(dst_hbm.at[pl.ds(off, _MCH)], dstv)

        def vec(j, carry2):
            d = dstv[pl.ds(16 * j, 16)]
            inb = (d >= lo) & (d < lo + _HALF)
            dlocv[pl.ds(16 * j, 16)] = jnp.where(inb, d - lo, _HALF)
            return carry2

        lax.fori_loop(0, _MCH // 16, vec, 0)
        pltpu.sync_copy(y_hbm.at[srcv], rows_v)
        pltpu.sync_copy(rows_v, shared.at[dlocv], add=True)
        return carry

    lax.fori_loop(0, _EPT // _MCH, chunk, 0)
    plsc.subcore_barrier()

    @pl.when(sid < 15)
    def _o0():
        pltpu.sync_copy(shared.at[pl.ds(sid * _ORC, _ORC)],
                        out_hbm.at[pl.ds(lo + sid * _ORC, _ORC)])

    @pl.when(sid == 15)
    def _o1():
        pltpu.sync_copy(shared.at[pl.ds(15 * _ORC, nz)],
                        out_hbm.at[pl.ds(lo + 15 * _ORC, nz)])


def _compute_accum(y, src, dst):
    zeros = jnp.zeros((_ORC, 256), jnp.float32)
    f = pl.kernel(
        _msg_body,
        out_type=jax.ShapeDtypeStruct((N, H), jnp.float32),
        mesh=_SC_MESH,
        compiler_params=pltpu.CompilerParams(use_tc_tiling_on_sc=False, needs_layout_passes=False),
        scratch_types=[
            pltpu.VMEM((_MCH,), jnp.int32),
            pltpu.VMEM((_MCH,), jnp.int32),
            pltpu.VMEM((_MCH,), jnp.int32),
            pltpu.VMEM((_MCH, H), jnp.float32),
            pltpu.VMEM_SHARED((_ACC_ROWS, H), jnp.float32),
        ],
    )
    return f(y, src, dst, zeros)


def _y_body(x_ref, wg_ref, deg0_ref, deg1_ref, y_ref, dinv_ref):
    deg = deg0_ref[...] + deg1_ref[...] + 1.0
    dinv = lax.rsqrt(deg)
    xw = jnp.dot(x_ref[...], wg_ref[...], preferred_element_type=jnp.float32)
    y_ref[...] = xw * dinv
    dinv_ref[...] = dinv


def _compute_y(x, wg, deg0, deg1):
    BR = 1000
    grid = (N // BR,)
    return pl.pallas_call(
        _y_body,
        grid=grid,
        in_specs=[
            pl.BlockSpec((BR, D), lambda i: (i, 0)),
            pl.BlockSpec((D, H), lambda i: (0, 0)),
            pl.BlockSpec((BR, 1), lambda i: (i, 0)),
            pl.BlockSpec((BR, 1), lambda i: (i, 0)),
        ],
        out_specs=[
            pl.BlockSpec((BR, H), lambda i: (i, 0)),
            pl.BlockSpec((BR, 1), lambda i: (i, 0)),
        ],
        out_shape=[
            jax.ShapeDtypeStruct((N, H), jnp.float32),
            jax.ShapeDtypeStruct((N, 1), jnp.float32),
        ],
    )(x, wg, deg0, deg1)


def _gx_body(acc_ref, y_ref, dinv_ref, bg_ref, wt_ref, bias_ref, gx_ref):
    xg = dinv_ref[...] * (acc_ref[...] + y_ref[...]) + bg_ref[...]
    gx_ref[...] = (
        jnp.dot(xg, wt_ref[...], preferred_element_type=jnp.float32) + bias_ref[...]
    )


def _compute_gx(accum, y, dinv, b_gcn, w_ihT, bias):
    BR = 1000
    grid = (N // BR,)
    return pl.pallas_call(
        _gx_body,
        grid=grid,
        in_specs=[
            pl.BlockSpec((BR, H), lambda i: (i, 0)),
            pl.BlockSpec((BR, H), lambda i: (i, 0)),
            pl.BlockSpec((BR, 1), lambda i: (i, 0)),
            pl.BlockSpec((1, H), lambda i: (0, 0)),
            pl.BlockSpec((H, G4), lambda i: (0, 0)),
            pl.BlockSpec((1, G4), lambda i: (0, 0)),
        ],
        out_specs=pl.BlockSpec((BR, G4), lambda i: (i, 0)),
        out_shape=jax.ShapeDtypeStruct((N, G4), jnp.float32),
    )(accum, y, dinv, b_gcn, w_ihT, bias)


def _lstm_body(gx_ref, whhT_ref, wlo_ref, h0_ref, c0_ref, out_ref, cn_ref, h_s, c_s, *, t_blk):
    @pl.when(pl.program_id(0) == 0)
    def _init():
        h_s[...] = h0_ref[...]
        c_s[...] = c0_ref[...]

    def step(t, carry):
        h, c = carry
        hb = h.astype(jnp.bfloat16)
        g = (
            jnp.dot(hb, whhT_ref[...], preferred_element_type=jnp.float32)
            + gx_ref[pl.ds(t, 1), :]
        )
        c_new = (jax.nn.sigmoid(g[:, H:2 * H]) * c
                 + jax.nn.sigmoid(g[:, 0:H]) * jnp.tanh(g[:, 2 * H:3 * H]))
        h_new = jax.nn.sigmoid(g[:, 3 * H:4 * H]) * jnp.tanh(c_new)
        out_ref[pl.ds(t, 1), :] = h_new
        return (h_new, c_new)

    h, c = lax.fori_loop(0, t_blk, step, (h_s[...], c_s[...]), unroll=8)
    h_s[...] = h
    c_s[...] = c
    cn_ref[...] = c


def _run_lstm(gx, w_hhT, h0, c0):
    w_hi = w_hhT.astype(jnp.bfloat16)
    w_lo = (w_hhT - w_hi.astype(jnp.float32)).astype(jnp.bfloat16)
    T_BLK = 400
    grid = (N // T_BLK,)
    return pl.pallas_call(
        functools.partial(_lstm_body, t_blk=T_BLK),
        grid=grid,
        in_specs=[
            pl.BlockSpec((T_BLK, G4), lambda i: (i, 0)),
            pl.BlockSpec((H, G4), lambda i: (0, 0)),
            pl.BlockSpec((H, G4), lambda i: (0, 0)),
            pl.BlockSpec((1, H), lambda i: (0, 0)),
            pl.BlockSpec((1, H), lambda i: (0, 0)),
        ],
        out_specs=[
            pl.BlockSpec((T_BLK, H), lambda i: (i, 0)),
            pl.BlockSpec((1, H), lambda i: (0, 0)),
        ],
        out_shape=[
            jax.ShapeDtypeStruct((N, H), jnp.float32),
            jax.ShapeDtypeStruct((1, H), jnp.float32),
        ],
        scratch_shapes=[
            pltpu.VMEM((1, H), jnp.float32),
            pltpu.VMEM((1, H), jnp.float32),
        ],
    )(gx, w_hi, w_lo, h0, c0)


def kernel(basic_block, edge_index, h0, c0, W_gcn, b_gcn, W_ih, W_hh, b_ih, b_hh):
    src = edge_index[0]
    dst = edge_index[1]

    degp = _compute_deg(dst)
    degp = degp.reshape(_NSC, _DR * 256)
    deg0 = degp[0, :N].reshape(N, 1)
    deg1 = degp[1, :N].reshape(N, 1)

    y, dinv = _compute_y(basic_block, W_gcn, deg0, deg1)

    accum = _compute_accum(y, src, dst)

    bias = (b_ih + b_hh).reshape(1, G4)
    gx = _compute_gx(accum, y, dinv, b_gcn.reshape(1, H), W_ih.T, bias)

    outs, c_n = _run_lstm(gx, W_hh.T, h0[0], c0[0])
    output = outs[:, None, :]
    h_n = outs[N - 1 :][None]
    c_n = c_n[None]
    return output, h_n, c_n


# speedup vs baseline: 14.6822x; 1.1018x over previous
import functools

import jax
import jax.numpy as jnp
from jax import lax
from jax.experimental import pallas as pl
from jax.experimental.pallas import tpu as pltpu
from jax.experimental.pallas import tpu_sc as plsc

N = 10000
E = 160000
D = 256
H = 256
G4 = 4 * H

_SC_MESH = plsc.VectorSubcoreMesh(core_axis_name="c", subcore_axis_name="s")
_NSC = 2
_NTILE = 16
_HALF = N // _NSC


_DR = 48
_EPW = E // (_NSC * _NTILE)
_DCH = 1000


def _deg_body(dst_hbm, zeros_hbm, out_hbm, idx_v, cnt_v, cnt2d_v, iota_v, shared):
    cid = lax.axis_index("c")
    sid = lax.axis_index("s")
    w = sid * _NSC + cid
    zeros16 = jnp.zeros((16,), jnp.float32)

    def zvec(k, carry):
        cnt_v[pl.ds(16 * k, 16)] = zeros16
        return carry

    lax.fori_loop(0, _DR * 16, zvec, 0)

    @pl.when(sid == 0)
    def _zero_shared():
        pltpu.sync_copy(zeros_hbm, shared)

    for j in range(_DR // 16):
        iota_v[pl.ds(16 * j, 16)] = lax.iota(jnp.int32, 16) + 16 * j
    plsc.subcore_barrier()

    ones = jnp.full((16,), 1.0, jnp.float32)
    tail_mask = lax.iota(jnp.int32, 16) < (_DCH % 16 or 16)
    base = w * _EPW

    def chunk(c, carry):
        pltpu.sync_copy(dst_hbm.at[pl.ds(base + c * _DCH, _DCH)],
                        idx_v.at[pl.ds(0, _DCH)])

        def vec(j, carry2):
            d = idx_v[pl.ds(16 * j, 16)]
            plsc.addupdate_scatter(cnt_v, [d], ones)
            return carry2

        lax.fori_loop(0, _DCH // 16, vec, 0)
        d = idx_v[pl.ds((_DCH // 16) * 16, 16)]
        plsc.addupdate_scatter(cnt_v, [d], ones, mask=tail_mask)
        return carry

    lax.fori_loop(0, _EPW // _DCH, chunk, 0)

    def pack(k, carry):
        r = k >> 4
        j = k & 15
        cnt2d_v[r, pl.ds(16 * j, 16)] = cnt_v[pl.ds(16 * k, 16)]
        return carry

    lax.fori_loop(0, _DR * 16, pack, 0)
    pltpu.sync_copy(cnt2d_v, shared.at[iota_v], add=True)
    plsc.subcore_barrier()

    @pl.when(sid == 0)
    def _out():
        pltpu.sync_copy(shared, out_hbm.at[cid])


def _compute_deg(dst):
    zeros = jnp.zeros((_DR, 256), jnp.float32)
    f = pl.kernel(
        _deg_body,
        out_type=jax.ShapeDtypeStruct((_NSC, _DR, 256), jnp.float32),
        mesh=_SC_MESH,
        compiler_params=pltpu.CompilerParams(use_tc_tiling_on_sc=False, needs_layout_passes=False),
        scratch_types=[
            pltpu.VMEM((_DCH + 8, ), jnp.int32),
            pltpu.VMEM((_DR * 256,), jnp.float32),
            pltpu.VMEM((_DR, 256), jnp.float32),
            pltpu.VMEM((_DR,), jnp.int32),
            pltpu.VMEM_SHARED((_DR, 256), jnp.float32),
        ],
    )
    return f(dst, zeros)


_ACC_ROWS = _HALF + 8
_MCH = 80
_EPT = E // _NTILE
_ORC = 320


def _msg_body(y_hbm, src_hbm, dst_hbm, zeros_hbm, out_hbm,
              srcv0, dstv0, dlocv0, rows0, srcv1, dstv1, dlocv1, rows1,
              shared, isem0, gsem0, isem1, gsem1):
    cid = lax.axis_index("c")
    sid = lax.axis_index("s")
    lo = cid * _HALF
    nz = _HALF - 15 * _ORC

    @pl.when(sid < 15)
    def _z0():
        pltpu.sync_copy(zeros_hbm, shared.at[pl.ds(sid * _ORC, _ORC)])

    @pl.when(sid == 15)
    def _z1():
        pltpu.sync_copy(zeros_hbm.at[pl.ds(0, nz)],
                        shared.at[pl.ds(15 * _ORC, nz)])

    plsc.subcore_barrier()
    base = sid * _EPT
    bufs = ((srcv0, dstv0, dlocv0, rows0, isem0, gsem0),
            (srcv1, dstv1, dlocv1, rows1, isem1, gsem1))
    nch = _EPT // _MCH

    def issue_idx(c, b):
        srcv, dstv, _, _, isem, _ = bufs[b]
        off = base + c * _MCH
        pltpu.async_copy(src_hbm.at[pl.ds(off, _MCH)], srcv, isem)
        pltpu.async_copy(dst_hbm.at[pl.ds(off, _MCH)], dstv, isem)

    def wait_idx(b):
        srcv, dstv, _, _, isem, _ = bufs[b]
        pltpu.make_async_copy(src_hbm.at[pl.ds(0, _MCH)], srcv, isem).wait()
        pltpu.make_async_copy(dst_hbm.at[pl.ds(0, _MCH)], dstv, isem).wait()

    def issue_gather(b):
        srcv, _, _, rows, _, gsem = bufs[b]
        pltpu.async_copy(y_hbm.at[srcv], rows, gsem)

    def wait_gather(b):
        srcv, _, _, rows, _, gsem = bufs[b]
        pltpu.make_async_copy(y_hbm.at[srcv], rows, gsem).wait()

    def remap(b):
        _, dstv, dlocv, _, _, _ = bufs[b]
        for j in range(_MCH // 16):
            d = dstv[pl.ds(16 * j, 16)]
            inb = (d >= lo) & (d < lo + _HALF)
            dlocv[pl.ds(16 * j, 16)] = jnp.where(inb, d - lo, _HALF)

    def scatter(b):
        _, _, dlocv, rows, _, _ = bufs[b]
        pltpu.sync_copy(rows, shared.at[dlocv], add=True)

    issue_idx(0, 0)
    wait_idx(0)
    issue_gather(0)
    issue_idx(1, 1)

    def pair(g, carry):
        c0 = 2 * g
        wait_idx(1)
        issue_gather(1)
        remap(0)
        wait_gather(0)
        issue_idx(c0 + 2, 0)
        scatter(0)
        remap(1)
        wait_gather(1)
        wait_idx(0)
        issue_gather(0)

        @pl.when(g < (nch - 1) // 2 - 1)
        def _pf():
            issue_idx(c0 + 3, 1)

        scatter(1)
        return carry

    lax.fori_loop(0, (nch - 1) // 2, pair, 0)
    remap(0)
    wait_gather(0)
    scatter(0)
    plsc.subcore_barrier()

    @pl.when(sid < 15)
    def _o0():
        pltpu.sync_copy(shared.at[pl.ds(sid * _ORC, _ORC)],
                        out_hbm.at[pl.ds(lo + sid * _ORC, _ORC)])

    @pl.when(sid == 15)
    def _o1():
        pltpu.sync_copy(shared.at[pl.ds(15 * _ORC, nz)],
                        out_hbm.at[pl.ds(lo + 15 * _ORC, nz)])


def _compute_accum(y, src, dst):
    zeros = jnp.zeros((_ORC, 256), jnp.float32)
    f = pl.kernel(
        _msg_body,
        out_type=jax.ShapeDtypeStruct((N, H), jnp.float32),
        mesh=_SC_MESH,
        compiler_params=pltpu.CompilerParams(use_tc_tiling_on_sc=False, needs_layout_passes=False),
        scratch_types=[
            pltpu.VMEM((_MCH,), jnp.int32),
            pltpu.VMEM((_MCH,), jnp.int32),
            pltpu.VMEM((_MCH,), jnp.int32),
            pltpu.VMEM((_MCH, H), jnp.float32),
            pltpu.VMEM((_MCH,), jnp.int32),
            pltpu.VMEM((_MCH,), jnp.int32),
            pltpu.VMEM((_MCH,), jnp.int32),
            pltpu.VMEM((_MCH, H), jnp.float32),
            pltpu.VMEM_SHARED((_ACC_ROWS, H), jnp.float32),
            pltpu.SemaphoreType.DMA,
            pltpu.SemaphoreType.DMA,
            pltpu.SemaphoreType.DMA,
            pltpu.SemaphoreType.DMA,
        ],
    )
    return f(y, src, dst, zeros)


def _y_body(x_ref, wg_ref, deg0_ref, deg1_ref, y_ref, dinv_ref):
    deg = deg0_ref[...] + deg1_ref[...] + 1.0
    dinv = lax.rsqrt(deg)
    xw = jnp.dot(x_ref[...], wg_ref[...], preferred_element_type=jnp.float32)
    y_ref[...] = xw * dinv
    dinv_ref[...] = dinv


def _compute_y(x, wg, deg0, deg1):
    BR = 1000
    grid = (N // BR,)
    return pl.pallas_call(
        _y_body,
        grid=grid,
        in_specs=[
            pl.BlockSpec((BR, D), lambda i: (i, 0)),
            pl.BlockSpec((D, H), lambda i: (0, 0)),
            pl.BlockSpec((BR, 1), lambda i: (i, 0)),
            pl.BlockSpec((BR, 1), lambda i: (i, 0)),
        ],
        out_specs=[
            pl.BlockSpec((BR, H), lambda i: (i, 0)),
            pl.BlockSpec((BR, 1), lambda i: (i, 0)),
        ],
        out_shape=[
            jax.ShapeDtypeStruct((N, H), jnp.float32),
            jax.ShapeDtypeStruct((N, 1), jnp.float32),
        ],
    )(x, wg, deg0, deg1)


def _gx_body(acc_ref, y_ref, dinv_ref, bg_ref, wt_ref, bias_ref, gx_ref):
    xg = dinv_ref[...] * (acc_ref[...] + y_ref[...]) + bg_ref[...]
    gx_ref[...] = (
        jnp.dot(xg, wt_ref[...], preferred_element_type=jnp.float32) + bias_ref[...]
    )


def _compute_gx(accum, y, dinv, b_gcn, w_ihT, bias):
    BR = 1000
    grid = (N // BR,)
    return pl.pallas_call(
        _gx_body,
        grid=grid,
        in_specs=[
            pl.BlockSpec((BR, H), lambda i: (i, 0)),
            pl.BlockSpec((BR, H), lambda i: (i, 0)),
            pl.BlockSpec((BR, 1), lambda i: (i, 0)),
            pl.BlockSpec((1, H), lambda i: (0, 0)),
            pl.BlockSpec((H, G4), lambda i: (0, 0)),
            pl.BlockSpec((1, G4), lambda i: (0, 0)),
        ],
        out_specs=pl.BlockSpec((BR, G4), lambda i: (i, 0)),
        out_shape=jax.ShapeDtypeStruct((N, G4), jnp.float32),
    )(accum, y, dinv, b_gcn, w_ihT, bias)


def _lstm_body(gx_ref, whhT_ref, wlo_ref, h0_ref, c0_ref, out_ref, cn_ref, h_s, c_s, *, t_blk):
    @pl.when(pl.program_id(0) == 0)
    def _init():
        h_s[...] = h0_ref[...]
        c_s[...] = c0_ref[...]

    def step(t, carry):
        h, c = carry
        hb = h.astype(jnp.bfloat16)
        g = (
            jnp.dot(hb, whhT_ref[...], preferred_element_type=jnp.float32)
            + gx_ref[pl.ds(t, 1), :]
        )
        c_new = (jax.nn.sigmoid(g[:, H:2 * H]) * c
                 + jax.nn.sigmoid(g[:, 0:H]) * jnp.tanh(g[:, 2 * H:3 * H]))
        h_new = jax.nn.sigmoid(g[:, 3 * H:4 * H]) * jnp.tanh(c_new)
        out_ref[pl.ds(t, 1), :] = h_new
        return (h_new, c_new)

    h, c = lax.fori_loop(0, t_blk, step, (h_s[...], c_s[...]), unroll=8)
    h_s[...] = h
    c_s[...] = c
    cn_ref[...] = c


def _run_lstm(gx, w_hhT, h0, c0):
    w_hi = w_hhT.astype(jnp.bfloat16)
    w_lo = (w_hhT - w_hi.astype(jnp.float32)).astype(jnp.bfloat16)
    T_BLK = 400
    grid = (N // T_BLK,)
    return pl.pallas_call(
        functools.partial(_lstm_body, t_blk=T_BLK),
        grid=grid,
        in_specs=[
            pl.BlockSpec((T_BLK, G4), lambda i: (i, 0)),
            pl.BlockSpec((H, G4), lambda i: (0, 0)),
            pl.BlockSpec((H, G4), lambda i: (0, 0)),
            pl.BlockSpec((1, H), lambda i: (0, 0)),
            pl.BlockSpec((1, H), lambda i: (0, 0)),
        ],
        out_specs=[
            pl.BlockSpec((T_BLK, H), lambda i: (i, 0)),
            pl.BlockSpec((1, H), lambda i: (0, 0)),
        ],
        out_shape=[
            jax.ShapeDtypeStruct((N, H), jnp.float32),
            jax.ShapeDtypeStruct((1, H), jnp.float32),
        ],
        scratch_shapes=[
            pltpu.VMEM((1, H), jnp.float32),
            pltpu.VMEM((1, H), jnp.float32),
        ],
    )(gx, w_hi, w_lo, h0, c0)


def kernel(basic_block, edge_index, h0, c0, W_gcn, b_gcn, W_ih, W_hh, b_ih, b_hh):
    src = edge_index[0]
    dst = edge_index[1]

    degp = _compute_deg(dst)
    degp = degp.reshape(_NSC, _DR * 256)
    deg0 = degp[0, :N].reshape(N, 1)
    deg1 = degp[1, :N].reshape(N, 1)

    y, dinv = _compute_y(basic_block, W_gcn, deg0, deg1)

    accum = _compute_accum(y, src, dst)

    bias = (b_ih + b_hh).reshape(1, G4)
    gx = _compute_gx(accum, y, dinv, b_gcn.reshape(1, H), W_ih.T, bias)

    outs, c_n = _run_lstm(gx, W_hh.T, h0[0], c0[0])
    output = outs[:, None, :]
    h_n = outs[N - 1 :][None]
    c_n = c_n[None]
    return output, h_n, c_n


# speedup vs baseline: 14.8726x; 1.0130x over previous
import functools

import jax
import jax.numpy as jnp
from jax import lax
from jax.experimental import pallas as pl
from jax.experimental.pallas import tpu as pltpu
from jax.experimental.pallas import tpu_sc as plsc

N = 10000
E = 160000
D = 256
H = 256
G4 = 4 * H

_SC_MESH = plsc.VectorSubcoreMesh(core_axis_name="c", subcore_axis_name="s")
_NSC = 2
_NTILE = 16
_HALF = N // _NSC


_DR = 48
_EPW = E // (_NSC * _NTILE)
_DCH = 1000


def _deg_body(dst_hbm, zeros_hbm, out_hbm, idx_v, cnt_v, cnt2d_v, iota_v, shared):
    cid = lax.axis_index("c")
    sid = lax.axis_index("s")
    w = sid * _NSC + cid
    zeros16 = jnp.zeros((16,), jnp.float32)

    def zvec(k, carry):
        cnt_v[pl.ds(16 * k, 16)] = zeros16
        return carry

    lax.fori_loop(0, _DR * 16, zvec, 0)

    @pl.when(sid == 0)
    def _zero_shared():
        pltpu.sync_copy(zeros_hbm, shared)

    for j in range(_DR // 16):
        iota_v[pl.ds(16 * j, 16)] = lax.iota(jnp.int32, 16) + 16 * j
    plsc.subcore_barrier()

    ones = jnp.full((16,), 1.0, jnp.float32)
    tail_mask = lax.iota(jnp.int32, 16) < (_DCH % 16 or 16)
    base = w * _EPW

    def chunk(c, carry):
        pltpu.sync_copy(dst_hbm.at[pl.ds(base + c * _DCH, _DCH)],
                        idx_v.at[pl.ds(0, _DCH)])

        def vec(j, carry2):
            d = idx_v[pl.ds(16 * j, 16)]
            plsc.addupdate_scatter(cnt_v, [d], ones)
            return carry2

        lax.fori_loop(0, _DCH // 16, vec, 0)
        d = idx_v[pl.ds((_DCH // 16) * 16, 16)]
        plsc.addupdate_scatter(cnt_v, [d], ones, mask=tail_mask)
        return carry

    lax.fori_loop(0, _EPW // _DCH, chunk, 0)

    def pack(k, carry):
        r = k >> 4
        j = k & 15
        cnt2d_v[r, pl.ds(16 * j, 16)] = cnt_v[pl.ds(16 * k, 16)]
        return carry

    lax.fori_loop(0, _DR * 16, pack, 0)
    pltpu.sync_copy(cnt2d_v, shared.at[iota_v], add=True)
    plsc.subcore_barrier()

    @pl.when(sid == 0)
    def _out():
        pltpu.sync_copy(shared, out_hbm.at[cid])


def _compute_deg(dst):
    zeros = jnp.zeros((_DR, 256), jnp.float32)
    f = pl.kernel(
        _deg_body,
        out_type=jax.ShapeDtypeStruct((_NSC, _DR, 256), jnp.float32),
        mesh=_SC_MESH,
        compiler_params=pltpu.CompilerParams(use_tc_tiling_on_sc=False, needs_layout_passes=False),
        scratch_types=[
            pltpu.VMEM((_DCH + 8, ), jnp.int32),
            pltpu.VMEM((_DR * 256,), jnp.float32),
            pltpu.VMEM((_DR, 256), jnp.float32),
            pltpu.VMEM((_DR,), jnp.int32),
            pltpu.VMEM_SHARED((_DR, 256), jnp.float32),
        ],
    )
    return f(dst, zeros)


_ACC_ROWS = _HALF + 8
_MCH = 80
_EPT = E // _NTILE
_ORC = 320


def _msg_body(y_hbm, src_hbm, dst_hbm, zeros_hbm, out_hbm,
              srcv0, dstv0, dlocv0, rows0, srcv1, dstv1, dlocv1, rows1,
              shared, isem0, gsem0, isem1, gsem1):
    cid = lax.axis_index("c")
    sid = lax.axis_index("s")
    lo = cid * _HALF
    nz = _HALF - 15 * _ORC

    @pl.when(sid < 15)
    def _z0():
        pltpu.sync_copy(zeros_hbm, shared.at[pl.ds(sid * _ORC, _ORC)])

    @pl.when(sid == 15)
    def _z1():
        pltpu.sync_copy(zeros_hbm.at[pl.ds(0, nz)],
                        shared.at[pl.ds(15 * _ORC, nz)])

    plsc.subcore_barrier()
    base = sid * _EPT
    bufs = ((srcv0, dstv0, dlocv0, rows0, isem0, gsem0),
            (srcv1, dstv1, dlocv1, rows1, isem1, gsem1))
    nch = _EPT // _MCH

    def issue_idx(c, b):
        srcv, dstv, _, _, isem, _ = bufs[b]
        off = base + c * _MCH
        pltpu.async_copy(src_hbm.at[pl.ds(off, _MCH)], srcv, isem)
        pltpu.async_copy(dst_hbm.at[pl.ds(off, _MCH)], dstv, isem)

    def wait_idx(b):
        srcv, dstv, _, _, isem, _ = bufs[b]
        pltpu.make_async_copy(src_hbm.at[pl.ds(0, _MCH)], srcv, isem).wait()
        pltpu.make_async_copy(dst_hbm.at[pl.ds(0, _MCH)], dstv, isem).wait()

    def issue_gather(b):
        srcv, _, _, rows, _, gsem = bufs[b]
        pltpu.async_copy(y_hbm.at[srcv], rows, gsem)

    def wait_gather(b):
        srcv, _, _, rows, _, gsem = bufs[b]
        pltpu.make_async_copy(y_hbm.at[srcv], rows, gsem).wait()

    def remap(b):
        _, dstv, dlocv, _, _, _ = bufs[b]
        for j in range(_MCH // 16):
            d = dstv[pl.ds(16 * j, 16)]
            inb = (d >= lo) & (d < lo + _HALF)
            dlocv[pl.ds(16 * j, 16)] = jnp.where(inb, d - lo, _HALF)

    def scatter(b):
        _, _, dlocv, rows, _, _ = bufs[b]
        pltpu.sync_copy(rows, shared.at[dlocv], add=True)

    issue_idx(0, 0)
    wait_idx(0)
    issue_gather(0)
    issue_idx(1, 1)

    def pair(g, carry):
        c0 = 2 * g
        wait_idx(1)
        issue_gather(1)
        remap(0)
        wait_gather(0)
        issue_idx(c0 + 2, 0)
        scatter(0)
        remap(1)
        wait_gather(1)
        wait_idx(0)
        issue_gather(0)

        @pl.when(g < (nch - 1) // 2 - 1)
        def _pf():
            issue_idx(c0 + 3, 1)

        scatter(1)
        return carry

    lax.fori_loop(0, (nch - 1) // 2, pair, 0)
    remap(0)
    wait_gather(0)
    scatter(0)
    plsc.subcore_barrier()

    @pl.when(sid < 15)
    def _o0():
        pltpu.sync_copy(shared.at[pl.ds(sid * _ORC, _ORC)],
                        out_hbm.at[pl.ds(lo + sid * _ORC, _ORC)])

    @pl.when(sid == 15)
    def _o1():
        pltpu.sync_copy(shared.at[pl.ds(15 * _ORC, nz)],
                        out_hbm.at[pl.ds(lo + 15 * _ORC, nz)])


def _compute_accum(y, src, dst):
    zeros = jnp.zeros((_ORC, 256), jnp.float32)
    f = pl.kernel(
        _msg_body,
        out_type=jax.ShapeDtypeStruct((N, H), jnp.float32),
        mesh=_SC_MESH,
        compiler_params=pltpu.CompilerParams(use_tc_tiling_on_sc=False, needs_layout_passes=False),
        scratch_types=[
            pltpu.VMEM((_MCH,), jnp.int32),
            pltpu.VMEM((_MCH,), jnp.int32),
            pltpu.VMEM((_MCH,), jnp.int32),
            pltpu.VMEM((_MCH, H), jnp.float32),
            pltpu.VMEM((_MCH,), jnp.int32),
            pltpu.VMEM((_MCH,), jnp.int32),
            pltpu.VMEM((_MCH,), jnp.int32),
            pltpu.VMEM((_MCH, H), jnp.float32),
            pltpu.VMEM_SHARED((_ACC_ROWS, H), jnp.float32),
            pltpu.SemaphoreType.DMA,
            pltpu.SemaphoreType.DMA,
            pltpu.SemaphoreType.DMA,
            pltpu.SemaphoreType.DMA,
        ],
    )
    return f(y, src, dst, zeros)


def _y_body(x_ref, wg_ref, deg0_ref, deg1_ref, y_ref, dinv_ref):
    deg = deg0_ref[...] + deg1_ref[...] + 1.0
    dinv = lax.rsqrt(deg)
    xw = jnp.dot(x_ref[...], wg_ref[...], preferred_element_type=jnp.float32)
    y_ref[...] = xw * dinv
    dinv_ref[...] = dinv


def _compute_y(x, wg, deg0, deg1):
    BR = 1000
    grid = (N // BR,)
    return pl.pallas_call(
        _y_body,
        grid=grid,
        in_specs=[
            pl.BlockSpec((BR, D), lambda i: (i, 0)),
            pl.BlockSpec((D, H), lambda i: (0, 0)),
            pl.BlockSpec((BR, 1), lambda i: (i, 0)),
            pl.BlockSpec((BR, 1), lambda i: (i, 0)),
        ],
        out_specs=[
            pl.BlockSpec((BR, H), lambda i: (i, 0)),
            pl.BlockSpec((BR, 1), lambda i: (i, 0)),
        ],
        out_shape=[
            jax.ShapeDtypeStruct((N, H), jnp.float32),
            jax.ShapeDtypeStruct((N, 1), jnp.float32),
        ],
    )(x, wg, deg0, deg1)


def _gx_body(acc_ref, y_ref, dinv_ref, bg_ref, wt_ref, bias_ref, gx_ref):
    xg = dinv_ref[...] * (acc_ref[...] + y_ref[...]) + bg_ref[...]
    gx_ref[...] = (
        jnp.dot(xg, wt_ref[...], preferred_element_type=jnp.float32) + bias_ref[...]
    )


def _compute_gx(accum, y, dinv, b_gcn, w_ihT, bias):
    BR = 1000
    grid = (N // BR,)
    return pl.pallas_call(
        _gx_body,
        grid=grid,
        in_specs=[
            pl.BlockSpec((BR, H), lambda i: (i, 0)),
            pl.BlockSpec((BR, H), lambda i: (i, 0)),
            pl.BlockSpec((BR, 1), lambda i: (i, 0)),
            pl.BlockSpec((1, H), lambda i: (0, 0)),
            pl.BlockSpec((H, G4), lambda i: (0, 0)),
            pl.BlockSpec((1, G4), lambda i: (0, 0)),
        ],
        out_specs=pl.BlockSpec((BR, G4), lambda i: (i, 0)),
        out_shape=jax.ShapeDtypeStruct((N, G4), jnp.float32),
    )(accum, y, dinv, b_gcn, w_ihT, bias)


def _lstm_body(gx_ref, whhT_ref, h0_ref, c0_ref, out_ref, cn_ref, h_s, c_s, *, t_blk):
    @pl.when(pl.program_id(0) == 0)
    def _init():
        h_s[...] = h0_ref[...]
        c_s[...] = c0_ref[...]

    def step(t, carry):
        h, c = carry
        hb = h.astype(jnp.bfloat16)
        g = (
            jnp.dot(hb, whhT_ref[...], preferred_element_type=jnp.float32)
            + gx_ref[pl.ds(t, 1), :]
        )
        c_new = (jax.nn.sigmoid(g[:, H:2 * H]) * c
                 + jax.nn.sigmoid(g[:, 0:H]) * jnp.tanh(g[:, 2 * H:3 * H]))
        h_new = jax.nn.sigmoid(g[:, 3 * H:4 * H]) * jnp.tanh(c_new)
        out_ref[pl.ds(t, 1), :] = h_new
        return (h_new, c_new)

    h, c = lax.fori_loop(0, t_blk, step, (h_s[...], c_s[...]), unroll=16)
    h_s[...] = h
    c_s[...] = c
    cn_ref[...] = c


def _run_lstm(gx, w_hhT, h0, c0):
    w_hi = w_hhT.astype(jnp.bfloat16)
    T_BLK = 400
    grid = (N // T_BLK,)
    return pl.pallas_call(
        functools.partial(_lstm_body, t_blk=T_BLK),
        grid=grid,
        in_specs=[
            pl.BlockSpec((T_BLK, G4), lambda i: (i, 0)),
            pl.BlockSpec((H, G4), lambda i: (0, 0)),
            pl.BlockSpec((1, H), lambda i: (0, 0)),
            pl.BlockSpec((1, H), lambda i: (0, 0)),
        ],
        out_specs=[
            pl.BlockSpec((T_BLK, H), lambda i: (i, 0)),
            pl.BlockSpec((1, H), lambda i: (0, 0)),
        ],
        out_shape=[
            jax.ShapeDtypeStruct((N, H), jnp.float32),
            jax.ShapeDtypeStruct((1, H), jnp.float32),
        ],
        scratch_shapes=[
            pltpu.VMEM((1, H), jnp.float32),
            pltpu.VMEM((1, H), jnp.float32),
        ],
    )(gx, w_hi, h0, c0)


def kernel(basic_block, edge_index, h0, c0, W_gcn, b_gcn, W_ih, W_hh, b_ih, b_hh):
    src = edge_index[0]
    dst = edge_index[1]

    degp = _compute_deg(dst)
    degp = degp.reshape(_NSC, _DR * 256)
    deg0 = degp[0, :N].reshape(N, 1)
    deg1 = degp[1, :N].reshape(N, 1)

    y, dinv = _compute_y(basic_block, W_gcn, deg0, deg1)

    accum = _compute_accum(y, src, dst)

    bias = (b_ih + b_hh).reshape(1, G4)
    gx = _compute_gx(accum, y, dinv, b_gcn.reshape(1, H), W_ih.T, bias)

    outs, c_n = _run_lstm(gx, W_hh.T, h0[0], c0[0])
    output = outs[:, None, :]
    h_n = outs[N - 1 :][None]
    c_n = c_n[None]
    return output, h_n, c_n


# speedup vs baseline: 14.9642x; 1.0062x over previous
import functools

import jax
import jax.numpy as jnp
from jax import lax
from jax.experimental import pallas as pl
from jax.experimental.pallas import tpu as pltpu
from jax.experimental.pallas import tpu_sc as plsc

N = 10000
E = 160000
D = 256
H = 256
G4 = 4 * H

_SC_MESH = plsc.VectorSubcoreMesh(core_axis_name="c", subcore_axis_name="s")
_NSC = 2
_NTILE = 16
_HALF = N // _NSC


_DR = 48
_EPW = E // (_NSC * _NTILE)
_DCH = 1000


def _deg_body(ei_hbm, zeros_hbm, out_hbm, idx_v, cnt_v, cnt2d_v, iota_v, shared):
    cid = lax.axis_index("c")
    sid = lax.axis_index("s")
    w = sid * _NSC + cid
    zeros16 = jnp.zeros((16,), jnp.float32)

    def zvec(k, carry):
        cnt_v[pl.ds(16 * k, 16)] = zeros16
        return carry

    lax.fori_loop(0, _DR * 16, zvec, 0)

    @pl.when(sid == 0)
    def _zero_shared():
        pltpu.sync_copy(zeros_hbm, shared)

    for j in range(_DR // 16):
        iota_v[pl.ds(16 * j, 16)] = lax.iota(jnp.int32, 16) + 16 * j
    plsc.subcore_barrier()

    ones = jnp.full((16,), 1.0, jnp.float32)
    tail_mask = lax.iota(jnp.int32, 16) < (_DCH % 16 or 16)
    base = w * _EPW

    def chunk(c, carry):
        pltpu.sync_copy(ei_hbm.at[1, pl.ds(base + c * _DCH, _DCH)],
                        idx_v.at[pl.ds(0, _DCH)])

        def vec(j, carry2):
            d = idx_v[pl.ds(16 * j, 16)]
            plsc.addupdate_scatter(cnt_v, [d], ones)
            return carry2

        lax.fori_loop(0, _DCH // 16, vec, 0)
        d = idx_v[pl.ds((_DCH // 16) * 16, 16)]
        plsc.addupdate_scatter(cnt_v, [d], ones, mask=tail_mask)
        return carry

    lax.fori_loop(0, _EPW // _DCH, chunk, 0)

    def pack(k, carry):
        r = k >> 4
        j = k & 15
        cnt2d_v[r, pl.ds(16 * j, 16)] = cnt_v[pl.ds(16 * k, 16)]
        return carry

    lax.fori_loop(0, _DR * 16, pack, 0)
    pltpu.sync_copy(cnt2d_v, shared.at[iota_v], add=True)
    plsc.subcore_barrier()

    @pl.when(sid == 0)
    def _out():
        pltpu.sync_copy(shared, out_hbm.at[cid])


def _compute_deg(edge_index):
    zeros = jnp.zeros((_DR, 256), jnp.float32)
    f = pl.kernel(
        _deg_body,
        out_type=jax.ShapeDtypeStruct((_NSC, _DR, 256), jnp.float32),
        mesh=_SC_MESH,
        compiler_params=pltpu.CompilerParams(use_tc_tiling_on_sc=False, needs_layout_passes=False),
        scratch_types=[
            pltpu.VMEM((_DCH + 8, ), jnp.int32),
            pltpu.VMEM((_DR * 256,), jnp.float32),
            pltpu.VMEM((_DR, 256), jnp.float32),
            pltpu.VMEM((_DR,), jnp.int32),
            pltpu.VMEM_SHARED((_DR, 256), jnp.float32),
        ],
    )
    return f(edge_index, zeros)


_ACC_ROWS = _HALF + 8
_MCH = 80
_EPT = E // _NTILE
_ORC = 320


def _msg_body(y_hbm, ei_hbm, zeros_hbm, out_hbm,
              srcv0, dstv0, dlocv0, rows0, srcv1, dstv1, dlocv1, rows1,
              shared, isem0, gsem0, isem1, gsem1):
    cid = lax.axis_index("c")
    sid = lax.axis_index("s")
    lo = cid * _HALF
    nz = _HALF - 15 * _ORC

    @pl.when(sid < 15)
    def _z0():
        pltpu.sync_copy(zeros_hbm, shared.at[pl.ds(sid * _ORC, _ORC)])

    @pl.when(sid == 15)
    def _z1():
        pltpu.sync_copy(zeros_hbm.at[pl.ds(0, nz)],
                        shared.at[pl.ds(15 * _ORC, nz)])

    plsc.subcore_barrier()
    base = sid * _EPT
    bufs = ((srcv0, dstv0, dlocv0, rows0, isem0, gsem0),
            (srcv1, dstv1, dlocv1, rows1, isem1, gsem1))
    nch = _EPT // _MCH

    def issue_idx(c, b):
        srcv, dstv, _, _, isem, _ = bufs[b]
        off = base + c * _MCH
        pltpu.async_copy(ei_hbm.at[0, pl.ds(off, _MCH)], srcv, isem)
        pltpu.async_copy(ei_hbm.at[1, pl.ds(off, _MCH)], dstv, isem)

    def wait_idx(b):
        srcv, dstv, _, _, isem, _ = bufs[b]
        pltpu.make_async_copy(ei_hbm.at[0, pl.ds(0, _MCH)], srcv, isem).wait()
        pltpu.make_async_copy(ei_hbm.at[1, pl.ds(0, _MCH)], dstv, isem).wait()

    def issue_gather(b):
        srcv, _, _, rows, _, gsem = bufs[b]
        pltpu.async_copy(y_hbm.at[srcv], rows, gsem)

    def wait_gather(b):
        srcv, _, _, rows, _, gsem = bufs[b]
        pltpu.make_async_copy(y_hbm.at[srcv], rows, gsem).wait()

    def remap(b):
        _, dstv, dlocv, _, _, _ = bufs[b]
        for j in range(_MCH // 16):
            d = dstv[pl.ds(16 * j, 16)]
            inb = (d >= lo) & (d < lo + _HALF)
            dlocv[pl.ds(16 * j, 16)] = jnp.where(inb, d - lo, _HALF)

    def scatter(b):
        _, _, dlocv, rows, _, _ = bufs[b]
        pltpu.sync_copy(rows, shared.at[dlocv], add=True)

    issue_idx(0, 0)
    wait_idx(0)
    issue_gather(0)
    issue_idx(1, 1)

    def pair(g, carry):
        c0 = 2 * g
        wait_idx(1)
        issue_gather(1)
        remap(0)
        wait_gather(0)
        issue_idx(c0 + 2, 0)
        scatter(0)
        remap(1)
        wait_gather(1)
        wait_idx(0)
        issue_gather(0)

        @pl.when(g < (nch - 1) // 2 - 1)
        def _pf():
            issue_idx(c0 + 3, 1)

        scatter(1)
        return carry

    lax.fori_loop(0, (nch - 1) // 2, pair, 0)
    remap(0)
    wait_gather(0)
    scatter(0)
    plsc.subcore_barrier()

    @pl.when(sid < 15)
    def _o0():
        pltpu.sync_copy(shared.at[pl.ds(sid * _ORC, _ORC)],
                        out_hbm.at[pl.ds(lo + sid * _ORC, _ORC)])

    @pl.when(sid == 15)
    def _o1():
        pltpu.sync_copy(shared.at[pl.ds(15 * _ORC, nz)],
                        out_hbm.at[pl.ds(lo + 15 * _ORC, nz)])


def _compute_accum(y, edge_index):
    zeros = jnp.zeros((_ORC, 256), jnp.float32)
    f = pl.kernel(
        _msg_body,
        out_type=jax.ShapeDtypeStruct((N, H), jnp.float32),
        mesh=_SC_MESH,
        compiler_params=pltpu.CompilerParams(use_tc_tiling_on_sc=False, needs_layout_passes=False),
        scratch_types=[
            pltpu.VMEM((_MCH,), jnp.int32),
            pltpu.VMEM((_MCH,), jnp.int32),
            pltpu.VMEM((_MCH,), jnp.int32),
            pltpu.VMEM((_MCH, H), jnp.float32),
            pltpu.VMEM((_MCH,), jnp.int32),
            pltpu.VMEM((_MCH,), jnp.int32),
            pltpu.VMEM((_MCH,), jnp.int32),
            pltpu.VMEM((_MCH, H), jnp.float32),
            pltpu.VMEM_SHARED((_ACC_ROWS, H), jnp.float32),
            pltpu.SemaphoreType.DMA,
            pltpu.SemaphoreType.DMA,
            pltpu.SemaphoreType.DMA,
            pltpu.SemaphoreType.DMA,
        ],
    )
    return f(y, edge_index, zeros)


def _y_body(x_ref, wg_ref, deg0_ref, deg1_ref, y_ref, dinv_ref):
    deg = deg0_ref[...] + deg1_ref[...] + 1.0
    dinv = lax.rsqrt(deg)
    xw = jnp.dot(x_ref[...], wg_ref[...], preferred_element_type=jnp.float32)
    y_ref[...] = xw * dinv
    dinv_ref[...] = dinv


def _compute_y(x, wg, deg0, deg1):
    BR = 1000
    grid = (N // BR,)
    return pl.pallas_call(
        _y_body,
        grid=grid,
        in_specs=[
            pl.BlockSpec((BR, D), lambda i: (i, 0)),
            pl.BlockSpec((D, H), lambda i: (0, 0)),
            pl.BlockSpec((BR, 1), lambda i: (i, 0)),
            pl.BlockSpec((BR, 1), lambda i: (i, 0)),
        ],
        out_specs=[
            pl.BlockSpec((BR, H), lambda i: (i, 0)),
            pl.BlockSpec((BR, 1), lambda i: (i, 0)),
        ],
        out_shape=[
            jax.ShapeDtypeStruct((N, H), jnp.float32),
            jax.ShapeDtypeStruct((N, 1), jnp.float32),
        ],
    )(x, wg, deg0, deg1)


def _gx_body(acc_ref, y_ref, dinv_ref, bg_ref, wt_ref, bias_ref, gx_ref):
    xg = dinv_ref[...] * (acc_ref[...] + y_ref[...]) + bg_ref[...]
    gx_ref[...] = (
        jnp.dot(xg, wt_ref[...], preferred_element_type=jnp.float32) + bias_ref[...]
    )


def _compute_gx(accum, y, dinv, b_gcn, w_ihT, bias):
    BR = 1000
    grid = (N // BR,)
    return pl.pallas_call(
        _gx_body,
        grid=grid,
        in_specs=[
            pl.BlockSpec((BR, H), lambda i: (i, 0)),
            pl.BlockSpec((BR, H), lambda i: (i, 0)),
            pl.BlockSpec((BR, 1), lambda i: (i, 0)),
            pl.BlockSpec((1, H), lambda i: (0, 0)),
            pl.BlockSpec((H, G4), lambda i: (0, 0)),
            pl.BlockSpec((1, G4), lambda i: (0, 0)),
        ],
        out_specs=pl.BlockSpec((BR, G4), lambda i: (i, 0)),
        out_shape=jax.ShapeDtypeStruct((N, G4), jnp.float32),
    )(accum, y, dinv, b_gcn, w_ihT, bias)


def _lstm_body(gx_ref, whhT_ref, h0_ref, c0_ref, out_ref, cn_ref, h_s, c_s, *, t_blk):
    @pl.when(pl.program_id(0) == 0)
    def _init():
        h_s[...] = h0_ref[...]
        c_s[...] = c0_ref[...]

    def step(t, carry):
        h, c = carry
        hb = h.astype(jnp.bfloat16)
        g = (
            jnp.dot(hb, whhT_ref[...], preferred_element_type=jnp.float32)
            + gx_ref[pl.ds(t, 1), :]
        )
        c_new = (jax.nn.sigmoid(g[:, H:2 * H]) * c
                 + jax.nn.sigmoid(g[:, 0:H]) * jnp.tanh(g[:, 2 * H:3 * H]))
        h_new = jax.nn.sigmoid(g[:, 3 * H:4 * H]) * jnp.tanh(c_new)
        out_ref[pl.ds(t, 1), :] = h_new
        return (h_new, c_new)

    h, c = lax.fori_loop(0, t_blk, step, (h_s[...], c_s[...]), unroll=16)
    h_s[...] = h
    c_s[...] = c
    cn_ref[...] = c


def _run_lstm(gx, w_hhT, h0, c0):
    w_hi = w_hhT.astype(jnp.bfloat16)
    T_BLK = 1000
    grid = (N // T_BLK,)
    return pl.pallas_call(
        functools.partial(_lstm_body, t_blk=T_BLK),
        grid=grid,
        in_specs=[
            pl.BlockSpec((T_BLK, G4), lambda i: (i, 0)),
            pl.BlockSpec((H, G4), lambda i: (0, 0)),
            pl.BlockSpec((1, H), lambda i: (0, 0)),
            pl.BlockSpec((1, H), lambda i: (0, 0)),
        ],
        out_specs=[
            pl.BlockSpec((T_BLK, H), lambda i: (i, 0)),
            pl.BlockSpec((1, H), lambda i: (0, 0)),
        ],
        out_shape=[
            jax.ShapeDtypeStruct((N, H), jnp.float32),
            jax.ShapeDtypeStruct((1, H), jnp.float32),
        ],
        scratch_shapes=[
            pltpu.VMEM((1, H), jnp.float32),
            pltpu.VMEM((1, H), jnp.float32),
        ],
    )(gx, w_hi, h0, c0)


def kernel(basic_block, edge_index, h0, c0, W_gcn, b_gcn, W_ih, W_hh, b_ih, b_hh):
    degp = _compute_deg(edge_index)
    degp = degp.reshape(_NSC, _DR * 256)
    deg0 = degp[0, :N].reshape(N, 1)
    deg1 = degp[1, :N].reshape(N, 1)

    y, dinv = _compute_y(basic_block, W_gcn, deg0, deg1)

    accum = _compute_accum(y, edge_index)

    bias = (b_ih + b_hh).reshape(1, G4)
    gx = _compute_gx(accum, y, dinv, b_gcn.reshape(1, H), W_ih.T, bias)

    outs, c_n = _run_lstm(gx, W_hh.T, h0[0], c0[0])
    output = outs[:, None, :]
    h_n = outs[N - 1 :][None]
    c_n = c_n[None]
    return output, h_n, c_n
